# baseline (speedup 1.0000x reference)
import functools
import math

import jax
import jax.numpy as jnp
from jax.experimental import pallas as pl
from jax.experimental.pallas import tpu as pltpu

D_MODEL_K = 4096
MAX_NORM_K = math.sqrt(D_MODEL_K)
BUF_ROWS = 1024
NBUF = 3


def _chunk_plan(n_batch, seq):
    ramp_up = [256, 256, 512]
    ramp_down = [512, 256, 256]
    chunks = []
    for b in range(n_batch):
        row0 = b * seq
        if b == 0:
            sizes = ramp_up + [BUF_ROWS] * ((seq - sum(ramp_up)) // BUF_ROWS)
        elif b == n_batch - 1:
            sizes = [BUF_ROWS] * ((seq - sum(ramp_down)) // BUF_ROWS) + ramp_down
        else:
            sizes = [BUF_ROWS] * (seq // BUF_ROWS)
        assert sum(sizes) == seq
        off = 0
        for sz in sizes:
            chunks.append((row0 + off, sz, b))
            off += sz
    return chunks


def _pipeline_kernel(ts_ref, x_hbm, tbl_hbm, o_hbm, buf, emb_ref,
                     in_sems, out_sems, row_sem, *, chunks, n_batch):
    for b in range(n_batch):
        pltpu.make_async_copy(
            tbl_hbm.at[pl.ds(ts_ref[b], 1), :], emb_ref.at[pl.ds(b, 1), :],
            row_sem,
        ).start()

    def copy_in(c, slot):
        row0, sz, _ = chunks[c]
        return pltpu.make_async_copy(
            x_hbm.at[pl.ds(row0, sz), :],
            buf.at[slot, pl.ds(0, sz), :],
            in_sems.at[slot],
        )

    def copy_out(c, slot):
        row0, sz, _ = chunks[c]
        return pltpu.make_async_copy(
            buf.at[slot, pl.ds(0, sz), :],
            o_hbm.at[pl.ds(row0, sz), :],
            out_sems.at[slot],
        )

    n_chunks = len(chunks)

    for s in range(min(NBUF, n_chunks)):
        copy_in(s, s).start()

    for b in range(n_batch):
        pltpu.make_async_copy(
            tbl_hbm.at[pl.ds(ts_ref[b], 1), :], emb_ref.at[pl.ds(b, 1), :],
            row_sem,
        ).wait()
    rows = emb_ref[...]
    norms = jnp.sqrt(jnp.sum(rows * rows, axis=-1, keepdims=True))
    emb_ref[...] = rows * jnp.where(norms > MAX_NORM_K,
                                    MAX_NORM_K / (norms + 1e-7), 1.0)

    for c in range(n_chunks):
        slot = c % NBUF
        _, sz, b = chunks[c]
        copy_in(c, slot).wait()
        buf[slot, pl.ds(0, sz), :] += emb_ref[pl.ds(b, 1), :]
        copy_out(c, slot).start()
        nxt = c + NBUF
        if nxt < n_chunks:
            copy_out(c, slot).wait()
            copy_in(nxt, slot).start()

    for c in range(max(0, n_chunks - NBUF), n_chunks):
        copy_out(c, c % NBUF).wait()


def kernel(x, timesteps, table):
    B, S, D = x.shape
    x2 = x.reshape(B * S, D)
    chunks = _chunk_plan(B, S)
    body = functools.partial(_pipeline_kernel, chunks=chunks, n_batch=B)
    out = pl.pallas_call(
        body,
        grid_spec=pltpu.PrefetchScalarGridSpec(
            num_scalar_prefetch=1,
            grid=(1,),
            in_specs=[
                pl.BlockSpec(memory_space=pl.ANY),
                pl.BlockSpec(memory_space=pl.ANY),
            ],
            out_specs=pl.BlockSpec(memory_space=pl.ANY),
            scratch_shapes=[
                pltpu.VMEM((NBUF, BUF_ROWS, D), x.dtype),
                pltpu.VMEM((B, D), x.dtype),
                pltpu.SemaphoreType.DMA((NBUF,)),
                pltpu.SemaphoreType.DMA((NBUF,)),
                pltpu.SemaphoreType.DMA,
            ],
        ),
        out_shape=jax.ShapeDtypeStruct(x2.shape, x.dtype),
    )(timesteps, x2, table)
    return out.reshape(B, S, D)


# speedup vs baseline: 1.0458x; 1.0458x over previous
import functools
import math

import jax
import jax.numpy as jnp
from jax.experimental import pallas as pl
from jax.experimental.pallas import tpu as pltpu

D_MODEL_K = 4096
MAX_NORM_K = math.sqrt(D_MODEL_K)
BUF_ROWS = 1024
NBUF = 3


def _chunk_plan(n_batch, seq):
    ramp_up = [1024]
    ramp_down = [1024]
    chunks = []
    for b in range(n_batch):
        row0 = b * seq
        if b == 0:
            sizes = ramp_up + [BUF_ROWS] * ((seq - sum(ramp_up)) // BUF_ROWS)
        elif b == n_batch - 1:
            sizes = [BUF_ROWS] * ((seq - sum(ramp_down)) // BUF_ROWS) + ramp_down
        else:
            sizes = [BUF_ROWS] * (seq // BUF_ROWS)
        assert sum(sizes) == seq
        off = 0
        for sz in sizes:
            chunks.append((row0 + off, sz, b))
            off += sz
    return chunks


def _pipeline_kernel(ts_ref, x_hbm, tbl_hbm, o_hbm, buf, emb_ref,
                     in_sems, out_sems, row_sem, *, chunks, n_batch):
    for b in range(n_batch):
        pltpu.make_async_copy(
            tbl_hbm.at[pl.ds(ts_ref[b], 1), :], emb_ref.at[pl.ds(b, 1), :],
            row_sem,
        ).start()

    def copy_in(c, slot):
        row0, sz, _ = chunks[c]
        return pltpu.make_async_copy(
            x_hbm.at[pl.ds(row0, sz), :],
            buf.at[slot, pl.ds(0, sz), :],
            in_sems.at[slot],
        )

    def copy_out(c, slot):
        row0, sz, _ = chunks[c]
        return pltpu.make_async_copy(
            buf.at[slot, pl.ds(0, sz), :],
            o_hbm.at[pl.ds(row0, sz), :],
            out_sems.at[slot],
        )

    n_chunks = len(chunks)

    for s in range(min(NBUF, n_chunks)):
        copy_in(s, s).start()

    for b in range(n_batch):
        pltpu.make_async_copy(
            tbl_hbm.at[pl.ds(ts_ref[b], 1), :], emb_ref.at[pl.ds(b, 1), :],
            row_sem,
        ).wait()
    rows = emb_ref[...]
    norms = jnp.sqrt(jnp.sum(rows * rows, axis=-1, keepdims=True))
    emb_ref[...] = rows * jnp.where(norms > MAX_NORM_K,
                                    MAX_NORM_K / (norms + 1e-7), 1.0)

    for c in range(n_chunks):
        slot = c % NBUF
        _, sz, b = chunks[c]
        copy_in(c, slot).wait()
        buf[slot, pl.ds(0, sz), :] += emb_ref[pl.ds(b, 1), :]
        copy_out(c, slot).start()
        nxt = c + NBUF
        if nxt < n_chunks:
            copy_out(c, slot).wait()
            copy_in(nxt, slot).start()

    for c in range(max(0, n_chunks - NBUF), n_chunks):
        copy_out(c, c % NBUF).wait()


def kernel(x, timesteps, table):
    B, S, D = x.shape
    x2 = x.reshape(B * S, D)
    chunks = _chunk_plan(B, S)
    body = functools.partial(_pipeline_kernel, chunks=chunks, n_batch=B)
    out = pl.pallas_call(
        body,
        grid_spec=pltpu.PrefetchScalarGridSpec(
            num_scalar_prefetch=1,
            grid=(1,),
            in_specs=[
                pl.BlockSpec(memory_space=pl.ANY),
                pl.BlockSpec(memory_space=pl.ANY),
            ],
            out_specs=pl.BlockSpec(memory_space=pl.ANY),
            scratch_shapes=[
                pltpu.VMEM((NBUF, BUF_ROWS, D), x.dtype),
                pltpu.VMEM((B, D), x.dtype),
                pltpu.SemaphoreType.DMA((NBUF,)),
                pltpu.SemaphoreType.DMA((NBUF,)),
                pltpu.SemaphoreType.DMA,
            ],
        ),
        out_shape=jax.ShapeDtypeStruct(x2.shape, x.dtype),
    )(timesteps, x2, table)
    return out.reshape(B, S, D)
